# baseline (speedup 1.0000x reference)
import functools

import jax
import jax.numpy as jnp
from jax import lax
from jax.experimental import pallas as pl
from jax.experimental.pallas import tpu as pltpu
from jax.experimental.pallas import tpu_sc as plsc

NX = 64
NSAMP = 1000000
BATCH = 16384
NUM_CORES = 2
NUM_SUBCORES = 16
NUM_WORKERS = NUM_CORES * NUM_SUBCORES
CROWS = NSAMP // 2

CV_CH = 160
CV_NCHG = NSAMP // CV_CH
CV_NBUF = 2

B_PER_W = BATCH // NUM_WORKERS
CH = 128
NCH = B_PER_W // CH


@functools.partial(
    pl.kernel,
    out_type=jax.ShapeDtypeStruct((CROWS, 2 * NX), jnp.float32),
    mesh=plsc.VectorSubcoreMesh(core_axis_name="c", subcore_axis_name="s"),
    scratch_types=[
        pltpu.VMEM((CV_NBUF, CV_CH, NX), jnp.float32),
        pltpu.VMEM((CV_NBUF, CV_CH // 2, 2 * NX), jnp.float32),
        pltpu.SemaphoreType.DMA,
        pltpu.SemaphoreType.DMA,
        pltpu.SemaphoreType.DMA,
        pltpu.SemaphoreType.DMA,
    ],
    compiler_params=pltpu.CompilerParams(use_tc_tiling_on_sc=True),
)
def _sc_convert(table_hbm, ctab_hbm, buf_v, pk_v, sem0, sem1, wsem0, wsem1):
    wid = lax.axis_index("s") * NUM_CORES + lax.axis_index("c")
    sems = (sem0, sem1)
    wsems = (wsem0, wsem1)

    def rd_start(cg, b):
        pltpu.make_async_copy(
            table_hbm.at[pl.ds(cg * CV_CH, CV_CH)],
            buf_v.at[b],
            sems[b],
        ).start()

    def rd_wait(b):
        pltpu.make_async_copy(
            table_hbm.at[pl.ds(0, CV_CH)], buf_v.at[b], sems[b]
        ).wait()

    def wr(cg, b):
        return pltpu.make_async_copy(
            pk_v.at[b],
            ctab_hbm.at[pl.ds(cg * (CV_CH // 2), CV_CH // 2)],
            wsems[b],
        )

    def repack(b):
        def rp(j, carry):
            for h in range(2):
                for k in range(NX // 16):
                    pk_v[b, j, pl.ds(h * NX + k * 16, 16)] = buf_v[
                        b, 2 * j + h, pl.ds(k * 16, 16)
                    ]
            return carry

        lax.fori_loop(0, CV_CH // 2, rp, 0)

    for b in range(CV_NBUF):
        rd_start(wid + b * NUM_WORKERS, b)

    def outer(g, carry):
        for b in range(CV_NBUF):
            k = g * CV_NBUF + b
            cg = wid + k * NUM_WORKERS

            @pl.when(cg < CV_NCHG)
            def _():
                rd_wait(b)

                @pl.when(k >= CV_NBUF)
                def _():
                    wr(0, b).wait()

                repack(b)
                wr(cg, b).start()

                @pl.when(cg + CV_NBUF * NUM_WORKERS < CV_NCHG)
                def _():
                    rd_start(cg + CV_NBUF * NUM_WORKERS, b)

        return carry

    niter = (CV_NCHG // NUM_WORKERS + CV_NBUF) // CV_NBUF + 1
    lax.fori_loop(0, niter, outer, 0)
    for b in range(CV_NBUF):
        last = wid + ((CV_NCHG - 1 - wid) // NUM_WORKERS) * NUM_WORKERS

        @pl.when(last >= 0)
        def _():
            wr(0, b).wait()


@functools.partial(
    pl.kernel,
    out_type=jax.ShapeDtypeStruct((BATCH, NX), jnp.float32),
    mesh=plsc.VectorSubcoreMesh(core_axis_name="c", subcore_axis_name="s"),
    scratch_types=[
        pltpu.VMEM((B_PER_W,), jnp.int32),
        pltpu.VMEM((B_PER_W,), jnp.int32),
        pltpu.VMEM((B_PER_W // 2, 2 * NX), jnp.float32),
        pltpu.VMEM((B_PER_W, NX), jnp.float32),
        pltpu.SemaphoreType.DMA,
    ],
    compiler_params=pltpu.CompilerParams(use_tc_tiling_on_sc=True),
)
def _sc_gather(ids_hbm, ctab_hbm, out_hbm, ids_v, pidx_v, pairs_v, rows_v,
               sem):
    wid = lax.axis_index("s") * NUM_CORES + lax.axis_index("c")
    base = wid * B_PER_W
    pltpu.sync_copy(ids_hbm.at[wid], ids_v)
    for s in range(B_PER_W // 16):
        sl = pl.ds(s * 16, 16)
        pidx_v[sl] = lax.shift_right_logical(ids_v[sl], 1)

    for half in range(2):
        hb = half * (B_PER_W // 2)
        copies = [
            pltpu.async_copy(
                ctab_hbm.at[pidx_v.at[pl.ds(hb + j * CH, CH)]],
                pairs_v.at[pl.ds(j * CH, CH)],
                sem,
            )
            for j in range(NCH // 2)
        ]
        for c in copies:
            c.wait()

        def pick(s, carry, hb=hb):
            vec = ids_v[pl.ds(hb + s * 16, 16)]
            for l in range(16):
                j = s * 16 + l
                off = lax.mul(lax.bitwise_and(vec[l], 1), NX)
                for k in range(NX // 16):
                    rows_v[hb + j, pl.ds(k * 16, 16)] = pairs_v[
                        j, pl.ds(off + k * 16, 16)
                    ]
            return carry

        lax.fori_loop(0, B_PER_W // 2 // 16, pick, 0)

    pltpu.sync_copy(rows_v, out_hbm.at[pl.ds(base, B_PER_W)])


def kernel(ids, start_state):
    ctab = _sc_convert(start_state)
    ids2 = ids.astype(jnp.int32).reshape(NUM_WORKERS, B_PER_W)
    return _sc_gather(ids2, ctab)


# speedup vs baseline: 2.2298x; 2.2298x over previous
import functools

import jax
import jax.numpy as jnp
from jax import lax
from jax.experimental import pallas as pl
from jax.experimental.pallas import tpu as pltpu
from jax.experimental.pallas import tpu_sc as plsc

NX = 64
BATCH = 16384
NUM_CORES = 2
NUM_SUBCORES = 16
NUM_WORKERS = NUM_CORES * NUM_SUBCORES
B_PER_W = BATCH // NUM_WORKERS


@functools.partial(
    pl.kernel,
    out_type=jax.ShapeDtypeStruct((BATCH, NX), jnp.float32),
    mesh=plsc.VectorSubcoreMesh(core_axis_name="c", subcore_axis_name="s"),
    scratch_types=[
        pltpu.VMEM((B_PER_W,), jnp.int32),
        pltpu.VMEM((B_PER_W, NX), jnp.float32),
        pltpu.SemaphoreType.DMA,
    ],
    compiler_params=pltpu.CompilerParams(use_tc_tiling_on_sc=True),
)
def _sc_gather(ids_hbm, table_hbm, out_hbm, ids_v, rows_v, sem):
    wid = lax.axis_index("s") * NUM_CORES + lax.axis_index("c")
    base = wid * B_PER_W
    pltpu.sync_copy(ids_hbm.at[wid], ids_v)

    def issue(s, carry):
        vec = ids_v[pl.ds(s * 16, 16)]
        for l in range(16):
            r = vec[l]
            pltpu.make_async_copy(
                table_hbm.at[r], rows_v.at[s * 16 + l], sem
            ).start()
        return carry

    lax.fori_loop(0, B_PER_W // 16, issue, 0)

    def drain(j, carry):
        pltpu.make_async_copy(table_hbm.at[0], rows_v.at[j], sem).wait()
        return carry

    lax.fori_loop(0, B_PER_W, drain, 0)

    pltpu.sync_copy(rows_v, out_hbm.at[pl.ds(base, B_PER_W)])


def kernel(ids, start_state):
    ids2 = ids.astype(jnp.int32).reshape(NUM_WORKERS, B_PER_W)
    return _sc_gather(ids2, start_state)
